# baseline (speedup 1.0000x reference)
import functools

import jax
import jax.numpy as jnp
from jax import lax
from jax.experimental import pallas as pl
from jax.experimental.pallas import tpu as pltpu
from jax.experimental.pallas import tpu_sc as plsc

N = 10000
E = 320000
D = 128
NC, NS, L = 2, 16, 16
NW = NC * NS
EPW = E // NW
C = 50
NCH = EPW // C
IBLK = 40
NBLK = NCH // IBLK
NBUF = 4
DC = 80
DNCH = EPW // DC
DIBLK = 25
DNBLK = DNCH // DIBLK
SEG = 40
NSEGT = N // SEG
DPAD = 10240
DPS = DPAD // NS
BN = 1000


_mesh = plsc.VectorSubcoreMesh(core_axis_name="c", subcore_axis_name="s")


@functools.partial(
    pl.kernel,
    out_type=jax.ShapeDtypeStruct((NC * DPAD,), jnp.float32),
    mesh=_mesh,
    scratch_types=[
        pltpu.VMEM((DNCH, DC), jnp.int32),
        pltpu.VMEM((DNCH, DC), jnp.float32),
        pltpu.VMEM((DPS,), jnp.float32),
        pltpu.VMEM_SHARED((DPAD,), jnp.float32),
        pltpu.SemaphoreType.DMA,
    ],
)
def _deg_kernel(dst_hbm, ew_hbm, out_hbm, dst_v, ew_v, zb, deg_sh, sem):
    cid = lax.axis_index("c")
    sid = lax.axis_index("s")
    wid = sid * NC + cid
    zeros = jnp.zeros((L,), jnp.float32)

    def zero_zb(i, carry):
        zb[pl.ds(i * L, L)] = zeros
        return carry

    lax.fori_loop(0, DPS // L, zero_zb, 0)
    pltpu.sync_copy(zb, deg_sh.at[pl.ds(sid * DPS, DPS)])
    plsc.subcore_barrier()

    pltpu.sync_copy(dst_hbm.at[wid], dst_v)
    pltpu.sync_copy(ew_hbm.at[wid], ew_v)

    def wave(w, carry):
        def fire(j, c2):
            pltpu.async_copy(ew_v.at[j], deg_sh.at[dst_v.at[j]], sem,
                             add=True)
            return c2

        lax.fori_loop(w * DIBLK, (w + 1) * DIBLK, fire, 0)

        def drain(j, c2):
            pltpu.make_async_copy(ew_v.at[0], deg_sh.at[dst_v.at[0]],
                                  sem).wait()
            return c2

        lax.fori_loop(0, DIBLK, drain, 0)
        return carry

    lax.fori_loop(0, DNBLK, wave, 0)
    plsc.subcore_barrier()

    pltpu.sync_copy(deg_sh.at[pl.ds(sid * DPS, DPS)], zb)
    pltpu.sync_copy(zb, out_hbm.at[pl.ds(cid * DPAD + sid * DPS, DPS)])


@functools.partial(
    pl.kernel,
    out_type=jax.ShapeDtypeStruct((NC, N, D), jnp.float32),
    mesh=_mesh,
    scratch_types=[
        pltpu.VMEM((IBLK, C), jnp.int32),
        pltpu.VMEM((IBLK, C), jnp.int32),
        pltpu.VMEM((IBLK, C), jnp.float32),
        [pltpu.VMEM((C, D // 2), jnp.int32) for _ in range(NBUF)],
        [pltpu.VMEM((C, D), jnp.float32) for _ in range(2)],
        pltpu.VMEM_SHARED((N, D), jnp.float32),
        [pltpu.SemaphoreType.DMA for _ in range(NBUF)],
        [pltpu.SemaphoreType.DMA for _ in range(2)],
    ],
    compiler_params=pltpu.CompilerParams(use_tc_tiling_on_sc=False),
)
def _acc_kernel(y_hbm, src_hbm, dst_hbm, ew_hbm, out_hbm,
                src_v, dst_v, ew_v, gb_bufs, f_bufs, acc_sh, gsems, ssems):
    cid = lax.axis_index("c")
    sid = lax.axis_index("s")
    wid = sid * NC + cid
    rows = f_bufs[0]
    zeros = jnp.zeros((L,), jnp.float32)

    def zero_rows(r, carry):
        for k in range(D // L):
            rows[r, pl.ds(k * L, L)] = zeros
        return carry

    lax.fori_loop(0, C, zero_rows, 0)
    for t in range(-(-NSEGT // NS)):
        seg = sid + NS * t

        @pl.when(seg < NSEGT)
        def _():
            pltpu.sync_copy(rows.at[pl.ds(0, SEG)],
                            acc_sh.at[pl.ds(seg * SEG, SEG)])

    plsc.subcore_barrier()

    himask = jnp.full((L,), -65536, jnp.int32)

    def scale(gb, fb, j):
        groups = [(g * L, range(L)) for g in range(C // L)]
        if C % L:
            groups.append((C - L, range(L - C % L, L)))
        for off, lanes in groups:
            ew16 = ew_v[j, pl.ds(off, L)]
            for i in lanes:
                r = off + i
                ewv = jnp.full((L,), ew16[i], jnp.float32)
                for k in range(D // (2 * L)):
                    w = gb[r, pl.ds(k * L, L)]
                    a = lax.bitcast_convert_type(w << 16, jnp.float32)
                    b2 = lax.bitcast_convert_type(w & himask, jnp.float32)
                    fb[r, pl.ds(k * 2 * L, L)] = a * ewv
                    fb[r, pl.ds(k * 2 * L + L, L)] = b2 * ewv

    def block(jb, carry):
        pltpu.sync_copy(src_hbm.at[wid, jb], src_v)
        pltpu.sync_copy(dst_hbm.at[wid, jb], dst_v)
        pltpu.sync_copy(ew_hbm.at[wid, jb], ew_v)
        pltpu.async_copy(y_hbm.at[src_v.at[0]], gb_bufs[0], gsems[0])
        pltpu.async_copy(y_hbm.at[src_v.at[1]], gb_bufs[1], gsems[1])

        def qstep(jj, c2):
            for b in range(NBUF):
                j = NBUF * jj + b
                rb, gs = gb_bufs[b], gsems[b]
                bn = (b + 2) % NBUF
                rn, gn = gb_bufs[bn], gsems[bn]
                fb, ss = f_bufs[b % 2], ssems[b % 2]
                pltpu.make_async_copy(y_hbm.at[src_v.at[j]], rb, gs).wait()

                @pl.when(j >= 2)
                def _():
                    pltpu.make_async_copy(fb, acc_sh.at[dst_v.at[j]],
                                          ss).wait()

                @pl.when(j + 2 < IBLK)
                def _():
                    pltpu.async_copy(y_hbm.at[src_v.at[j + 2]], rn, gn)

                scale(rb, fb, j)
                pltpu.async_copy(fb, acc_sh.at[dst_v.at[j]], ss, add=True)
            return c2

        lax.fori_loop(0, IBLK // NBUF, qstep, 0)
        pltpu.make_async_copy(f_bufs[0], acc_sh.at[dst_v.at[0]],
                              ssems[0]).wait()
        pltpu.make_async_copy(f_bufs[1], acc_sh.at[dst_v.at[0]],
                              ssems[1]).wait()
        return carry

    lax.fori_loop(0, NBLK, block, 0)
    plsc.subcore_barrier()

    for t in range(-(-NSEGT // NS)):
        seg = sid + NS * t

        @pl.when(seg < NSEGT)
        def _():
            sl = pl.ds(seg * SEG, SEG)
            pltpu.sync_copy(acc_sh.at[sl], rows.at[pl.ds(0, SEG)])
            pltpu.sync_copy(rows.at[pl.ds(0, SEG)], out_hbm.at[cid, sl])


def _p2_body(x_ref, w_ref, d0_ref, d1_ref, y_ref, dis_ref):
    deg = d0_ref[...] + d1_ref[...] + 1.0
    dis = lax.rsqrt(deg)
    dis_ref[...] = dis
    y_ref[...] = jnp.dot(x_ref[...], w_ref[...],
                         preferred_element_type=jnp.float32) * dis


def _p4_body(a0_ref, a1_ref, y1_ref, dis_ref, b1_ref, w2_ref, y2_ref):
    dis = dis_ref[...]
    h = jnp.maximum(
        (a0_ref[...] + a1_ref[...] + y1_ref[...]) * dis + b1_ref[...], 0.0)
    y2_ref[...] = jnp.dot(h, w2_ref[...],
                          preferred_element_type=jnp.float32) * dis


def _p6_body(a0_ref, a1_ref, y2_ref, dis_ref, b2_ref, o_ref):
    o_ref[...] = ((a0_ref[...] + a1_ref[...] + y2_ref[...]) * dis_ref[...]
                  + b2_ref[...])


_row_spec = pl.BlockSpec((BN, D), lambda i: (i, 0))
_col_spec = pl.BlockSpec((BN, 1), lambda i: (i, 0))
_w_spec = pl.BlockSpec((D, D), lambda i: (0, 0))
_b_spec = pl.BlockSpec((1, D), lambda i: (0, 0))

_p2 = pl.pallas_call(
    _p2_body,
    grid=(N // BN,),
    in_specs=[_row_spec, _w_spec, _col_spec, _col_spec],
    out_specs=[_row_spec, _col_spec],
    out_shape=[
        jax.ShapeDtypeStruct((N, D), jnp.float32),
        jax.ShapeDtypeStruct((N, 1), jnp.float32),
    ],
)

_p4 = pl.pallas_call(
    _p4_body,
    grid=(N // BN,),
    in_specs=[_row_spec, _row_spec, _row_spec, _col_spec, _b_spec, _w_spec],
    out_specs=_row_spec,
    out_shape=jax.ShapeDtypeStruct((N, D), jnp.float32),
)

_p6 = pl.pallas_call(
    _p6_body,
    grid=(N // BN,),
    in_specs=[_row_spec, _row_spec, _row_spec, _col_spec, _b_spec],
    out_specs=_row_spec,
    out_shape=jax.ShapeDtypeStruct((N, D), jnp.float32),
)


def kernel(x, edge_index, edge_weight, W1, b1, W2, b2):
    src = edge_index[0].astype(jnp.int32).reshape(NW, NBLK, IBLK, C)
    dst = edge_index[1].astype(jnp.int32).reshape(NW, NBLK, IBLK, C)
    ew = edge_weight.reshape(NW, NBLK, IBLK, C)

    deg_parts = _deg_kernel(
        dst.reshape(NW, DNCH, DC), ew.reshape(NW, DNCH, DC)).reshape(NC, DPAD)
    d0 = deg_parts[0, :N].reshape(N, 1)
    d1 = deg_parts[1, :N].reshape(N, 1)

    def permute_bf16(y):
        yb = (y.reshape(N, D // 32, 2, 16).transpose(0, 1, 3, 2)
              .reshape(N, D // 2, 2).astype(jnp.bfloat16))
        return lax.bitcast_convert_type(yb, jnp.int32)

    y1, dis = _p2(x, W1, d0, d1)
    acc1 = _acc_kernel(permute_bf16(y1), src, dst, ew)
    y2 = _p4(acc1[0], acc1[1], y1, dis, b1.reshape(1, D), W2)
    acc2 = _acc_kernel(permute_bf16(y2), src, dst, ew)
    out = _p6(acc2[0], acc2[1], y2, dis, b2.reshape(1, D))
    return out


# speedup vs baseline: 1.0723x; 1.0723x over previous
import functools

import jax
import jax.numpy as jnp
from jax import lax
from jax.experimental import pallas as pl
from jax.experimental.pallas import tpu as pltpu
from jax.experimental.pallas import tpu_sc as plsc

N = 10000
E = 320000
D = 128
NC, NS, L = 2, 16, 16
NW = NC * NS
EPW = E // NW
C = 50
NCH = EPW // C
IBLK = 40
NBLK = NCH // IBLK
NBUF = 4
DC = 80
DNCH = EPW // DC
DIBLK = 25
DNBLK = DNCH // DIBLK
SEG = 40
NSEGT = N // SEG
DPAD = 10240
DPS = DPAD // NS
BN = 1000


_mesh = plsc.VectorSubcoreMesh(core_axis_name="c", subcore_axis_name="s")


@functools.partial(
    pl.kernel,
    out_type=jax.ShapeDtypeStruct((NC * DPAD,), jnp.float32),
    mesh=_mesh,
    scratch_types=[
        pltpu.VMEM((DNCH, DC), jnp.int32),
        pltpu.VMEM((DNCH, DC), jnp.float32),
        pltpu.VMEM((DPS,), jnp.float32),
        pltpu.VMEM_SHARED((DPAD,), jnp.float32),
        pltpu.SemaphoreType.DMA,
    ],
)
def _deg_kernel(dst_hbm, ew_hbm, out_hbm, dst_v, ew_v, zb, deg_sh, sem):
    cid = lax.axis_index("c")
    sid = lax.axis_index("s")
    wid = sid * NC + cid
    zeros = jnp.zeros((L,), jnp.float32)

    def zero_zb(i, carry):
        zb[pl.ds(i * L, L)] = zeros
        return carry

    lax.fori_loop(0, DPS // L, zero_zb, 0)
    pltpu.sync_copy(zb, deg_sh.at[pl.ds(sid * DPS, DPS)])
    plsc.subcore_barrier()

    pltpu.sync_copy(dst_hbm.at[wid], dst_v)
    pltpu.sync_copy(ew_hbm.at[wid], ew_v)

    def wave(w, carry):
        def fire(j, c2):
            pltpu.async_copy(ew_v.at[j], deg_sh.at[dst_v.at[j]], sem,
                             add=True)
            return c2

        lax.fori_loop(w * DIBLK, (w + 1) * DIBLK, fire, 0)

        def drain(j, c2):
            pltpu.make_async_copy(ew_v.at[0], deg_sh.at[dst_v.at[0]],
                                  sem).wait()
            return c2

        lax.fori_loop(0, DIBLK, drain, 0)
        return carry

    lax.fori_loop(0, DNBLK, wave, 0)
    plsc.subcore_barrier()

    pltpu.sync_copy(deg_sh.at[pl.ds(sid * DPS, DPS)], zb)
    pltpu.sync_copy(zb, out_hbm.at[pl.ds(cid * DPAD + sid * DPS, DPS)])


@functools.partial(
    pl.kernel,
    out_type=jax.ShapeDtypeStruct((NC, N, D), jnp.float32),
    mesh=_mesh,
    scratch_types=[
        pltpu.VMEM((IBLK, C), jnp.int32),
        pltpu.VMEM((IBLK, C), jnp.int32),
        pltpu.VMEM((IBLK, C), jnp.float32),
        [pltpu.VMEM((C, D), jnp.float32) for _ in range(NBUF)],
        pltpu.VMEM_SHARED((N, D), jnp.float32),
        [pltpu.SemaphoreType.DMA for _ in range(NBUF)],
        [pltpu.SemaphoreType.DMA for _ in range(NBUF)],
    ],
)
def _acc_kernel(y_hbm, src_hbm, dst_hbm, ew_hbm, out_hbm,
                src_v, dst_v, ew_v, rows_bufs, acc_sh, gsems, ssems):
    cid = lax.axis_index("c")
    sid = lax.axis_index("s")
    wid = sid * NC + cid
    rows = rows_bufs[0]
    zeros = jnp.zeros((L,), jnp.float32)

    def zero_rows(r, carry):
        for k in range(D // L):
            rows[r, pl.ds(k * L, L)] = zeros
        return carry

    lax.fori_loop(0, C, zero_rows, 0)
    for t in range(-(-NSEGT // NS)):
        seg = sid + NS * t

        @pl.when(seg < NSEGT)
        def _():
            pltpu.sync_copy(rows.at[pl.ds(0, SEG)],
                            acc_sh.at[pl.ds(seg * SEG, SEG)])

    plsc.subcore_barrier()

    def scale(rows_ref, j):
        groups = [(g * L, range(L)) for g in range(C // L)]
        if C % L:
            groups.append((C - L, range(L - C % L, L)))
        for off, lanes in groups:
            ew16 = ew_v[j, pl.ds(off, L)]
            for i in lanes:
                r = off + i
                ewv = jnp.full((L,), ew16[i], jnp.float32)
                for k in range(D // L):
                    rows_ref[r, pl.ds(k * L, L)] = (
                        rows_ref[r, pl.ds(k * L, L)] * ewv)

    def block(jb, carry):
        pltpu.sync_copy(src_hbm.at[wid, jb], src_v)
        pltpu.sync_copy(dst_hbm.at[wid, jb], dst_v)
        pltpu.sync_copy(ew_hbm.at[wid, jb], ew_v)
        pltpu.async_copy(y_hbm.at[src_v.at[0]], rows_bufs[0], gsems[0])
        pltpu.async_copy(y_hbm.at[src_v.at[1]], rows_bufs[1], gsems[1])

        def qstep(jj, c2):
            for b in range(NBUF):
                j = NBUF * jj + b
                rb, gs, ss = rows_bufs[b], gsems[b], ssems[b]
                bn = (b + 2) % NBUF
                rn, gn, sn = rows_bufs[bn], gsems[bn], ssems[bn]
                pltpu.make_async_copy(y_hbm.at[src_v.at[j]], rb, gs).wait()

                @pl.when(j >= 2)
                def _():
                    pltpu.make_async_copy(rn, acc_sh.at[dst_v.at[j]],
                                          sn).wait()

                @pl.when(j + 2 < IBLK)
                def _():
                    pltpu.async_copy(y_hbm.at[src_v.at[j + 2]], rn, gn)

                scale(rb, j)
                pltpu.async_copy(rb, acc_sh.at[dst_v.at[j]], ss, add=True)
            return c2

        lax.fori_loop(0, IBLK // NBUF, qstep, 0)
        pltpu.make_async_copy(rows_bufs[2], acc_sh.at[dst_v.at[0]],
                              ssems[2]).wait()
        pltpu.make_async_copy(rows_bufs[3], acc_sh.at[dst_v.at[0]],
                              ssems[3]).wait()
        return carry

    lax.fori_loop(0, NBLK, block, 0)
    plsc.subcore_barrier()

    for t in range(-(-NSEGT // NS)):
        seg = sid + NS * t
        buf = rows_bufs[t % 2].at[pl.ds(0, SEG)]

        @pl.when(seg < NSEGT)
        def _():
            sl = pl.ds(seg * SEG, SEG)

            @pl.when(t >= 2)
            def _():
                pltpu.make_async_copy(buf, out_hbm.at[cid, sl],
                                      gsems[t % 2]).wait()

            pltpu.sync_copy(acc_sh.at[sl], buf)
            pltpu.async_copy(buf, out_hbm.at[cid, sl], gsems[t % 2])

    for b in range(2):
        pltpu.make_async_copy(rows_bufs[b].at[pl.ds(0, SEG)],
                              out_hbm.at[cid, pl.ds(0, SEG)],
                              gsems[b]).wait()


def _p2_body(x_ref, w_ref, d0_ref, d1_ref, y_ref, dis_ref):
    deg = d0_ref[...] + d1_ref[...] + 1.0
    dis = lax.rsqrt(deg)
    dis_ref[...] = dis
    y_ref[...] = jnp.dot(x_ref[...], w_ref[...],
                         preferred_element_type=jnp.float32) * dis


def _p4_body(a0_ref, a1_ref, y1_ref, dis_ref, b1_ref, w2_ref, y2_ref):
    dis = dis_ref[...]
    h = jnp.maximum(
        (a0_ref[...] + a1_ref[...] + y1_ref[...]) * dis + b1_ref[...], 0.0)
    y2_ref[...] = jnp.dot(h, w2_ref[...],
                          preferred_element_type=jnp.float32) * dis


def _p6_body(a0_ref, a1_ref, y2_ref, dis_ref, b2_ref, o_ref):
    o_ref[...] = ((a0_ref[...] + a1_ref[...] + y2_ref[...]) * dis_ref[...]
                  + b2_ref[...])


_row_spec = pl.BlockSpec((BN, D), lambda i: (i, 0))
_col_spec = pl.BlockSpec((BN, 1), lambda i: (i, 0))
_w_spec = pl.BlockSpec((D, D), lambda i: (0, 0))
_b_spec = pl.BlockSpec((1, D), lambda i: (0, 0))

_p2 = pl.pallas_call(
    _p2_body,
    grid=(N // BN,),
    in_specs=[_row_spec, _w_spec, _col_spec, _col_spec],
    out_specs=[_row_spec, _col_spec],
    out_shape=[
        jax.ShapeDtypeStruct((N, D), jnp.float32),
        jax.ShapeDtypeStruct((N, 1), jnp.float32),
    ],
)

_p4 = pl.pallas_call(
    _p4_body,
    grid=(N // BN,),
    in_specs=[_row_spec, _row_spec, _row_spec, _col_spec, _b_spec, _w_spec],
    out_specs=_row_spec,
    out_shape=jax.ShapeDtypeStruct((N, D), jnp.float32),
)

_p6 = pl.pallas_call(
    _p6_body,
    grid=(N // BN,),
    in_specs=[_row_spec, _row_spec, _row_spec, _col_spec, _b_spec],
    out_specs=_row_spec,
    out_shape=jax.ShapeDtypeStruct((N, D), jnp.float32),
)


def kernel(x, edge_index, edge_weight, W1, b1, W2, b2):
    src = edge_index[0].astype(jnp.int32).reshape(NW, NBLK, IBLK, C)
    dst = edge_index[1].astype(jnp.int32).reshape(NW, NBLK, IBLK, C)
    ew = edge_weight.reshape(NW, NBLK, IBLK, C)

    deg_parts = _deg_kernel(
        dst.reshape(NW, DNCH, DC), ew.reshape(NW, DNCH, DC)).reshape(NC, DPAD)
    d0 = deg_parts[0, :N].reshape(N, 1)
    d1 = deg_parts[1, :N].reshape(N, 1)

    y1, dis = _p2(x, W1, d0, d1)
    acc1 = _acc_kernel(y1, src, dst, ew)
    y2 = _p4(acc1[0], acc1[1], y1, dis, b1.reshape(1, D), W2)
    acc2 = _acc_kernel(y2, src, dst, ew)
    out = _p6(acc2[0], acc2[1], y2, dis, b2.reshape(1, D))
    return out


# speedup vs baseline: 1.1078x; 1.0331x over previous
import functools

import jax
import jax.numpy as jnp
from jax import lax
from jax.experimental import pallas as pl
from jax.experimental.pallas import tpu as pltpu
from jax.experimental.pallas import tpu_sc as plsc

N = 10000
E = 320000
D = 128
NC, NS, L = 2, 16, 16
NW = NC * NS
EPW = E // NW
C = 50
NCH = EPW // C
IBLK = 40
NBLK = NCH // IBLK
NBUF = 4
DC = 80
DNCH = EPW // DC
DIBLK = 25
DNBLK = DNCH // DIBLK
SEG = 40
NSEGT = N // SEG
DPAD = 10240
DPS = DPAD // NS
BN = 1000


_mesh = plsc.VectorSubcoreMesh(core_axis_name="c", subcore_axis_name="s")


@functools.partial(
    pl.kernel,
    out_type=jax.ShapeDtypeStruct((NC * DPAD,), jnp.float32),
    mesh=_mesh,
    scratch_types=[
        pltpu.VMEM((DNCH, DC), jnp.int32),
        pltpu.VMEM((DNCH, DC), jnp.float32),
        pltpu.VMEM((DPS,), jnp.float32),
        pltpu.VMEM_SHARED((DPAD,), jnp.float32),
        pltpu.SemaphoreType.DMA,
    ],
)
def _deg_kernel(dst_hbm, ew_hbm, out_hbm, dst_v, ew_v, zb, deg_sh, sem):
    cid = lax.axis_index("c")
    sid = lax.axis_index("s")
    wid = sid * NC + cid
    zeros = jnp.zeros((L,), jnp.float32)

    def zero_zb(i, carry):
        zb[pl.ds(i * L, L)] = zeros
        return carry

    lax.fori_loop(0, DPS // L, zero_zb, 0)
    pltpu.sync_copy(zb, deg_sh.at[pl.ds(sid * DPS, DPS)])
    plsc.subcore_barrier()

    pltpu.sync_copy(dst_hbm.at[wid], dst_v)
    pltpu.sync_copy(ew_hbm.at[wid], ew_v)

    def wave(w, carry):
        def fire(j, c2):
            pltpu.async_copy(ew_v.at[j], deg_sh.at[dst_v.at[j]], sem,
                             add=True)
            return c2

        lax.fori_loop(w * DIBLK, (w + 1) * DIBLK, fire, 0)

        def drain(j, c2):
            pltpu.make_async_copy(ew_v.at[0], deg_sh.at[dst_v.at[0]],
                                  sem).wait()
            return c2

        lax.fori_loop(0, DIBLK, drain, 0)
        return carry

    lax.fori_loop(0, DNBLK, wave, 0)
    plsc.subcore_barrier()

    pltpu.sync_copy(deg_sh.at[pl.ds(sid * DPS, DPS)], zb)
    pltpu.sync_copy(zb, out_hbm.at[pl.ds(cid * DPAD + sid * DPS, DPS)])


@functools.partial(
    pl.kernel,
    out_type=jax.ShapeDtypeStruct((NC, N, D), jnp.float32),
    mesh=_mesh,
    scratch_types=[
        pltpu.VMEM((IBLK, C), jnp.int32),
        pltpu.VMEM((IBLK, C), jnp.int32),
        pltpu.VMEM((IBLK, C), jnp.float32),
        [pltpu.VMEM((C, D), jnp.float32) for _ in range(NBUF)],
        pltpu.VMEM_SHARED((N, D), jnp.float32),
        [pltpu.SemaphoreType.DMA for _ in range(NBUF)],
        [pltpu.SemaphoreType.DMA for _ in range(NBUF)],
    ],
)
def _acc_kernel(y_hbm, src_hbm, dst_hbm, ew_hbm, out_hbm,
                src_v, dst_v, ew_v, rows_bufs, acc_sh, gsems, ssems):
    cid = lax.axis_index("c")
    sid = lax.axis_index("s")
    wid = sid * NC + cid
    rows = rows_bufs[0]
    zeros = jnp.zeros((L,), jnp.float32)

    def zero_rows(r, carry):
        for k in range(D // L):
            rows[r, pl.ds(k * L, L)] = zeros
        return carry

    lax.fori_loop(0, C, zero_rows, 0)
    for t in range(-(-NSEGT // NS)):
        seg = sid + NS * t

        @pl.when(seg < NSEGT)
        def _():
            pltpu.sync_copy(rows.at[pl.ds(0, SEG)],
                            acc_sh.at[pl.ds(seg * SEG, SEG)])

    plsc.subcore_barrier()

    def scale(rows_ref, j):
        groups = [(g * L, range(L)) for g in range(C // L)]
        if C % L:
            groups.append((C - L, range(L - C % L, L)))
        for off, lanes in groups:
            ew16 = ew_v[j, pl.ds(off, L)]
            for i in lanes:
                r = off + i
                ewv = jnp.full((L,), ew16[i], jnp.float32)
                for k in range(D // L):
                    rows_ref[r, pl.ds(k * L, L)] = (
                        rows_ref[r, pl.ds(k * L, L)] * ewv)

    def block(jb, carry):
        pltpu.async_copy(src_hbm.at[wid, jb], src_v, gsems[2])
        pltpu.async_copy(dst_hbm.at[wid, jb], dst_v, gsems[3])
        pltpu.async_copy(ew_hbm.at[wid, jb], ew_v, ssems[2])
        pltpu.make_async_copy(src_hbm.at[wid, jb], src_v, gsems[2]).wait()
        pltpu.make_async_copy(dst_hbm.at[wid, jb], dst_v, gsems[3]).wait()
        pltpu.make_async_copy(ew_hbm.at[wid, jb], ew_v, ssems[2]).wait()
        pltpu.async_copy(y_hbm.at[src_v.at[0]], rows_bufs[0], gsems[0])
        pltpu.async_copy(y_hbm.at[src_v.at[1]], rows_bufs[1], gsems[1])

        def qstep(jj, c2):
            for b in range(NBUF):
                j = NBUF * jj + b
                rb, gs, ss = rows_bufs[b], gsems[b], ssems[b]
                bn = (b + 2) % NBUF
                rn, gn, sn = rows_bufs[bn], gsems[bn], ssems[bn]
                pltpu.make_async_copy(y_hbm.at[src_v.at[j]], rb, gs).wait()

                @pl.when(j >= 2)
                def _():
                    pltpu.make_async_copy(rn, acc_sh.at[dst_v.at[j]],
                                          sn).wait()

                @pl.when(j + 2 < IBLK)
                def _():
                    pltpu.async_copy(y_hbm.at[src_v.at[j + 2]], rn, gn)

                scale(rb, j)
                pltpu.async_copy(rb, acc_sh.at[dst_v.at[j]], ss, add=True)
            return c2

        lax.fori_loop(0, IBLK // NBUF, qstep, 0)
        pltpu.make_async_copy(rows_bufs[2], acc_sh.at[dst_v.at[0]],
                              ssems[2]).wait()
        pltpu.make_async_copy(rows_bufs[3], acc_sh.at[dst_v.at[0]],
                              ssems[3]).wait()
        return carry

    lax.fori_loop(0, NBLK, block, 0)
    plsc.subcore_barrier()

    for t in range(-(-NSEGT // NS)):
        seg = sid + NS * t
        buf = rows_bufs[t % 2].at[pl.ds(0, SEG)]

        @pl.when(seg < NSEGT)
        def _():
            sl = pl.ds(seg * SEG, SEG)

            @pl.when(t >= 2)
            def _():
                pltpu.make_async_copy(buf, out_hbm.at[cid, sl],
                                      gsems[t % 2]).wait()

            pltpu.sync_copy(acc_sh.at[sl], buf)
            pltpu.async_copy(buf, out_hbm.at[cid, sl], gsems[t % 2])

    for b in range(2):
        pltpu.make_async_copy(rows_bufs[b].at[pl.ds(0, SEG)],
                              out_hbm.at[cid, pl.ds(0, SEG)],
                              gsems[b]).wait()


def _p2_body(x_ref, w_ref, d0_ref, d1_ref, y_ref, dis_ref):
    deg = d0_ref[...] + d1_ref[...] + 1.0
    dis = lax.rsqrt(deg)
    dis_ref[...] = dis
    y_ref[...] = jnp.dot(x_ref[...], w_ref[...],
                         preferred_element_type=jnp.float32) * dis


def _p4_body(a0_ref, a1_ref, y1_ref, dis_ref, b1_ref, w2_ref, y2_ref):
    dis = dis_ref[...]
    h = jnp.maximum(
        (a0_ref[...] + a1_ref[...] + y1_ref[...]) * dis + b1_ref[...], 0.0)
    y2_ref[...] = jnp.dot(h, w2_ref[...],
                          preferred_element_type=jnp.float32) * dis


def _p6_body(a0_ref, a1_ref, y2_ref, dis_ref, b2_ref, o_ref):
    o_ref[...] = ((a0_ref[...] + a1_ref[...] + y2_ref[...]) * dis_ref[...]
                  + b2_ref[...])


_row_spec = pl.BlockSpec((BN, D), lambda i: (i, 0))
_col_spec = pl.BlockSpec((BN, 1), lambda i: (i, 0))
_w_spec = pl.BlockSpec((D, D), lambda i: (0, 0))
_b_spec = pl.BlockSpec((1, D), lambda i: (0, 0))

_p2 = pl.pallas_call(
    _p2_body,
    grid=(N // BN,),
    in_specs=[_row_spec, _w_spec, _col_spec, _col_spec],
    out_specs=[_row_spec, _col_spec],
    out_shape=[
        jax.ShapeDtypeStruct((N, D), jnp.float32),
        jax.ShapeDtypeStruct((N, 1), jnp.float32),
    ],
)

_p4 = pl.pallas_call(
    _p4_body,
    grid=(N // BN,),
    in_specs=[_row_spec, _row_spec, _row_spec, _col_spec, _b_spec, _w_spec],
    out_specs=_row_spec,
    out_shape=jax.ShapeDtypeStruct((N, D), jnp.float32),
)

_p6 = pl.pallas_call(
    _p6_body,
    grid=(N // BN,),
    in_specs=[_row_spec, _row_spec, _row_spec, _col_spec, _b_spec],
    out_specs=_row_spec,
    out_shape=jax.ShapeDtypeStruct((N, D), jnp.float32),
)


def kernel(x, edge_index, edge_weight, W1, b1, W2, b2):
    src = edge_index[0].astype(jnp.int32).reshape(NW, NBLK, IBLK, C)
    dst = edge_index[1].astype(jnp.int32).reshape(NW, NBLK, IBLK, C)
    ew = edge_weight.reshape(NW, NBLK, IBLK, C)

    deg_parts = _deg_kernel(
        dst.reshape(NW, DNCH, DC), ew.reshape(NW, DNCH, DC)).reshape(NC, DPAD)
    d0 = deg_parts[0, :N].reshape(N, 1)
    d1 = deg_parts[1, :N].reshape(N, 1)

    y1, dis = _p2(x, W1, d0, d1)
    acc1 = _acc_kernel(y1, src, dst, ew)
    y2 = _p4(acc1[0], acc1[1], y1, dis, b1.reshape(1, D), W2)
    acc2 = _acc_kernel(y2, src, dst, ew)
    out = _p6(acc2[0], acc2[1], y2, dis, b2.reshape(1, D))
    return out
